# baseline (speedup 1.0000x reference)
import jax
import jax.numpy as jnp
from jax import lax
from jax.experimental import pallas as pl
from jax.experimental.pallas import tpu as pltpu

ROWS = 16384
COLS = 1000
M = 200
MPAD = 208
BR = 1024
NCH = ROWS // BR
KIN = 8
KOUT = 2
RIN = BR // KIN
ROUT = BR // KOUT


def _body(mask_ref, x_hbm, o_hbm, w_ref, xv0, xv1, ov0, ov1, sin, sout):
    colid = lax.broadcasted_iota(jnp.int32, (COLS, MPAD), 0)
    mrow = jnp.broadcast_to(mask_ref[...], (COLS, MPAD))
    w_ref[...] = (colid == mrow).astype(jnp.bfloat16)

    xvs = (xv0, xv1)
    ovs = (ov0, ov1)

    def start_in(g):
        b = g % 2
        hs = []
        for k in range(KIN):
            h = pltpu.make_async_copy(
                x_hbm.at[pl.ds(g * BR + k * RIN, RIN)],
                xvs[b].at[pl.ds(k * RIN, RIN)],
                sin.at[b, k])
            h.start()
            hs.append(h)
        return hs

    def start_out(g):
        b = g % 2
        hs = []
        for k in range(KOUT):
            h = pltpu.make_async_copy(
                ovs[b].at[pl.ds(k * ROUT, ROUT)],
                o_hbm.at[pl.ds(g * BR + k * ROUT, ROUT)],
                sout.at[b, k])
            h.start()
            hs.append(h)
        return hs

    in_h = [None] * NCH
    out_h = [None] * NCH

    in_h[0] = start_in(0)

    for g in range(NCH):
        b = g % 2
        if g + 1 < NCH:
            in_h[g + 1] = start_in(g + 1)
        for h in in_h[g]:
            h.wait()
        if g >= 2:
            for h in out_h[g - 2]:
                h.wait()

        xb = xvs[b][...].astype(jnp.bfloat16)
        res = lax.dot_general(xb, w_ref[...], (((1,), (0,)), ((), ())),
                              preferred_element_type=jnp.float32)
        ovs[b][...] = res[:, :M]

        out_h[g] = start_out(g)

    for h in out_h[NCH - 2]:
        h.wait()
    for h in out_h[NCH - 1]:
        h.wait()


def kernel(x, mask):
    mask2 = jnp.concatenate(
        [mask, jnp.zeros((MPAD - M,), jnp.int32)]).reshape(1, MPAD)
    return pl.pallas_call(
        _body,
        in_specs=[
            pl.BlockSpec((1, MPAD), memory_space=pltpu.VMEM),
            pl.BlockSpec(memory_space=pl.ANY),
        ],
        out_specs=pl.BlockSpec(memory_space=pl.ANY),
        out_shape=jax.ShapeDtypeStruct((ROWS, M), jnp.float32),
        scratch_shapes=[
            pltpu.VMEM((COLS, MPAD), jnp.bfloat16),
            pltpu.VMEM((BR, COLS), jnp.float32),
            pltpu.VMEM((BR, COLS), jnp.float32),
            pltpu.VMEM((BR, M), jnp.float32),
            pltpu.VMEM((BR, M), jnp.float32),
            pltpu.SemaphoreType.DMA((2, KIN)),
            pltpu.SemaphoreType.DMA((2, KOUT)),
        ],
    )(mask2, x)


# speedup vs baseline: 1.0276x; 1.0276x over previous
import jax
import jax.numpy as jnp
from jax import lax
from jax.experimental import pallas as pl
from jax.experimental.pallas import tpu as pltpu

ROWS = 16384
COLS = 1000
M = 200
MPAD = 208
BR = 4096
NCH = ROWS // BR
KIN = 1
KOUT = 1
RIN = BR // KIN
ROUT = BR // KOUT


def _body(mask_ref, x_hbm, o_hbm, w_ref, xv0, xv1, ov0, ov1, sin, sout):
    colid = lax.broadcasted_iota(jnp.int32, (COLS, MPAD), 0)
    mrow = jnp.broadcast_to(mask_ref[...], (COLS, MPAD))
    w_ref[...] = (colid == mrow).astype(jnp.bfloat16)

    xvs = (xv0, xv1)
    ovs = (ov0, ov1)

    def start_in(g):
        b = g % 2
        hs = []
        for k in range(KIN):
            h = pltpu.make_async_copy(
                x_hbm.at[pl.ds(g * BR + k * RIN, RIN)],
                xvs[b].at[pl.ds(k * RIN, RIN)],
                sin.at[b, k])
            h.start()
            hs.append(h)
        return hs

    def start_out(g):
        b = g % 2
        hs = []
        for k in range(KOUT):
            h = pltpu.make_async_copy(
                ovs[b].at[pl.ds(k * ROUT, ROUT)],
                o_hbm.at[pl.ds(g * BR + k * ROUT, ROUT)],
                sout.at[b, k])
            h.start()
            hs.append(h)
        return hs

    in_h = [None] * NCH
    out_h = [None] * NCH

    in_h[0] = start_in(0)

    for g in range(NCH):
        b = g % 2
        if g + 1 < NCH:
            in_h[g + 1] = start_in(g + 1)
        for h in in_h[g]:
            h.wait()
        if g >= 2:
            for h in out_h[g - 2]:
                h.wait()

        xb = xvs[b][...].astype(jnp.bfloat16)
        res = lax.dot_general(xb, w_ref[...], (((1,), (0,)), ((), ())),
                              preferred_element_type=jnp.float32)
        ovs[b][...] = res[:, :M]

        out_h[g] = start_out(g)

    for h in out_h[NCH - 2]:
        h.wait()
    for h in out_h[NCH - 1]:
        h.wait()


def kernel(x, mask):
    mask2 = jnp.concatenate(
        [mask, jnp.zeros((MPAD - M,), jnp.int32)]).reshape(1, MPAD)
    return pl.pallas_call(
        _body,
        in_specs=[
            pl.BlockSpec((1, MPAD), memory_space=pltpu.VMEM),
            pl.BlockSpec(memory_space=pl.ANY),
        ],
        out_specs=pl.BlockSpec(memory_space=pl.ANY),
        out_shape=jax.ShapeDtypeStruct((ROWS, M), jnp.float32),
        scratch_shapes=[
            pltpu.VMEM((COLS, MPAD), jnp.bfloat16),
            pltpu.VMEM((BR, COLS), jnp.float32),
            pltpu.VMEM((BR, COLS), jnp.float32),
            pltpu.VMEM((BR, M), jnp.float32),
            pltpu.VMEM((BR, M), jnp.float32),
            pltpu.SemaphoreType.DMA((2, KIN)),
            pltpu.SemaphoreType.DMA((2, KOUT)),
        ],
    )(mask2, x)
